# Initial kernel scaffold; baseline (speedup 1.0000x reference)
import functools

import jax
import jax.numpy as jnp
from jax import lax
from jax.experimental import pallas as pl
from jax.experimental.pallas import tpu as pltpu
from jax.experimental.pallas import tpu_sc as plsc

N = 10000
E = 320000
M = 3
H = 4
DH = 64
DIN = 128
F = H * DH
FH = F // 2

NC = 2
NS = 16
NW = NC * NS
RPT = N // NS

TW = M * E // NW
KE = 120
NCHUNK_E = TW // KE

EPT = E // NS
KA = 80
NCHUNK_A = EPT // KA

BN = 2000
NB = N // BN

_mesh = plsc.VectorSubcoreMesh(
    core_axis_name="c", subcore_axis_name="s", num_cores=NC, num_subcores=NS)


@functools.partial(
    pl.kernel,
    out_type=jax.ShapeDtypeStruct((M * E * 4,), jnp.float32),
    mesh=_mesh,
    scratch_types=[
        pltpu.VMEM((KE,), jnp.int32),
        pltpu.VMEM((KE,), jnp.int32),
        pltpu.VMEM((KE, 16), jnp.float32),
        pltpu.VMEM((KE, 16), jnp.float32),
        pltpu.VMEM((KE * 4,), jnp.float32),
        pltpu.SemaphoreType.DMA,
        pltpu.SemaphoreType.DMA,
    ],
)
def _edge_ex(stab_hbm, gsrc_hbm, gdst_hbm, ex_hbm,
             sidx, didx, sv, dv, exv, sem1, sem2):
    wid = lax.axis_index("s") * NC + lax.axis_index("c")
    base0 = wid * TW

    def chunk(i, carry):
        b = base0 + i * KE
        pltpu.sync_copy(gsrc_hbm.at[pl.ds(b, KE)], sidx)
        pltpu.sync_copy(gdst_hbm.at[pl.ds(b, KE)], didx)
        cp1 = pltpu.async_copy(stab_hbm.at[sidx], sv, sem1)
        cp2 = pltpu.async_copy(stab_hbm.at[didx], dv, sem2)
        cp1.wait()
        cp2.wait()

        def step(t, c2):
            j = t * 16 + lax.iota(jnp.int32, 16)
            e = lax.shift_right_logical(j, 2)
            hh = lax.bitwise_and(j, 3)
            a = plsc.load_gather(sv, [e, hh])
            bb = plsc.load_gather(dv, [e, hh + 4])
            s = a + bb
            s = jnp.maximum(s, 0.2 * s)
            exv[pl.ds(t * 16, 16)] = jnp.exp(s)
            return c2

        lax.fori_loop(0, KE * 4 // 16, step, 0)
        pltpu.sync_copy(exv, ex_hbm.at[pl.ds(b * 4, KE * 4)])
        return carry

    lax.fori_loop(0, NCHUNK_E, chunk, 0)


@functools.partial(
    pl.kernel,
    out_type=(
        jax.ShapeDtypeStruct((NC, M, N, FH), jnp.float32),
        jax.ShapeDtypeStruct((M, N, H), jnp.float32),
    ),
    mesh=_mesh,
    scratch_types=[
        pltpu.VMEM((RPT, FH), jnp.float32),
        pltpu.VMEM((RPT, H), jnp.float32),
        pltpu.VMEM((KA, FH), jnp.float32),
        pltpu.VMEM((KA,), jnp.int32),
        pltpu.VMEM((KA,), jnp.int32),
        pltpu.VMEM((KA,), jnp.int32),
        pltpu.VMEM((KA, H), jnp.float32),
        pltpu.VMEM_SHARED((N, FH), jnp.float32),
        pltpu.VMEM_SHARED((N, H), jnp.float32),
        pltpu.SemaphoreType.DMA,
    ],
)
def _agg(hp_cm, gsrc_hbm, dst_hbm, ex_hbm, z128_hbm, z4_hbm,
         agg_out, den_out,
         zbuf, zd, rows, sidx, sidx2, didx, exb, acc, dacc, sem):
    cid = lax.axis_index("c")
    sid = lax.axis_index("s")
    r0 = sid * RPT
    coff = cid * (M * N)
    c2 = cid * 2

    for m in range(M):
        pltpu.sync_copy(z128_hbm, zbuf)
        pltpu.sync_copy(zbuf, acc.at[pl.ds(r0, RPT)])

        @pl.when(cid == 0)
        def _():
            pltpu.sync_copy(z4_hbm, zd)
            pltpu.sync_copy(zd, dacc.at[pl.ds(r0, RPT)])

        plsc.subcore_barrier()

        def chunk(i, carry):
            b = sid * EPT + i * KA
            pltpu.sync_copy(gsrc_hbm.at[m, pl.ds(b, KA)], sidx)
            pltpu.sync_copy(dst_hbm.at[m, pl.ds(b, KA)], didx)
            pltpu.sync_copy(ex_hbm.at[m, pl.ds(b, KA)], exb)
            for t in range(KA // 16):
                sidx2[pl.ds(t * 16, 16)] = sidx[pl.ds(t * 16, 16)] + coff
            pltpu.async_copy(hp_cm.at[sidx2], rows, sem).wait()

            def scale(e2, cc):
                s0 = exb[e2, c2]
                s1 = exb[e2, c2 + 1]
                for j in range(4):
                    rows[e2, pl.ds(j * 16, 16)] = rows[e2, pl.ds(j * 16, 16)] * s0
                for j in range(4, 8):
                    rows[e2, pl.ds(j * 16, 16)] = rows[e2, pl.ds(j * 16, 16)] * s1
                return cc

            lax.fori_loop(0, KA, scale, 0)
            pltpu.sync_copy(rows, acc.at[didx], add=True)

            @pl.when(cid == 0)
            def _():
                pltpu.sync_copy(exb, dacc.at[didx], add=True)

            return carry

        lax.fori_loop(0, NCHUNK_A, chunk, 0)
        plsc.subcore_barrier()

        pltpu.sync_copy(acc.at[pl.ds(r0, RPT)], zbuf)
        pltpu.sync_copy(zbuf, agg_out.at[cid, m, pl.ds(r0, RPT), :])

        @pl.when(cid == 0)
        def _():
            pltpu.sync_copy(dacc.at[pl.ds(r0, RPT)], zd)
            pltpu.sync_copy(zd, den_out.at[m, pl.ds(r0, RPT), :])

        plsc.subcore_barrier()


def _dense_body(x_ref, w_ref, smat_ref, hph_ref, stab_ref):
    xb = x_ref[...]
    hp = jnp.dot(xb, w_ref[0], preferred_element_type=jnp.float32)
    hph_ref[0, 0] = hp[:, :FH]
    hph_ref[1, 0] = hp[:, FH:]
    stab_ref[0] = jnp.dot(hp, smat_ref[0], preferred_element_type=jnp.float32)


def _make_dense(din):
    return pl.pallas_call(
        _dense_body,
        grid=(M, NB),
        in_specs=[
            pl.BlockSpec((BN, din), lambda m, i: (i, 0)),
            pl.BlockSpec((1, din, F), lambda m, i: (m, 0, 0)),
            pl.BlockSpec((1, F, 16), lambda m, i: (m, 0, 0)),
        ],
        out_specs=[
            pl.BlockSpec((NC, 1, BN, FH), lambda m, i: (0, m, i, 0)),
            pl.BlockSpec((1, BN, 16), lambda m, i: (m, i, 0)),
        ],
        out_shape=[
            jax.ShapeDtypeStruct((NC, M, N, FH), jnp.float32),
            jax.ShapeDtypeStruct((M, N, 16), jnp.float32),
        ],
    )


_dense0 = _make_dense(DIN)
_dense1 = _make_dense(DH)


def _elu(v):
    return jnp.where(v > 0, v, jnp.exp(v) - 1.0)


def _mc_body(aggh_ref, den_ref, q_ref, h_ref):
    zs = []
    for m in range(M):
        acc = None
        for h in range(H):
            c, off = h // 2, (h % 2) * DH
            v = aggh_ref[c, m, :, off:off + DH] / (den_ref[m, :, h:h + 1] + 1e-9)
            ev = _elu(v)
            acc = ev if acc is None else acc + ev
        zs.append(acc * (1.0 / H))
    q = q_ref[...]
    ss = [jnp.sum(jnp.tanh(z) * q, axis=1, keepdims=True) for z in zs]
    smax = jnp.maximum(jnp.maximum(ss[0], ss[1]), ss[2])
    es = [jnp.exp(s - smax) for s in ss]
    tot = es[0] + es[1] + es[2]
    hsum = sum((e / tot) * z for e, z in zip(es, zs))
    h_ref[...] = jnp.maximum(hsum, 0.0)


_mc = pl.pallas_call(
    _mc_body,
    grid=(NB,),
    in_specs=[
        pl.BlockSpec((NC, M, BN, FH), lambda i: (0, 0, i, 0)),
        pl.BlockSpec((M, BN, H), lambda i: (0, i, 0)),
        pl.BlockSpec((1, DH), lambda i: (0, 0)),
    ],
    out_specs=pl.BlockSpec((BN, DH), lambda i: (i, 0)),
    out_shape=jax.ShapeDtypeStruct((N, DH), jnp.float32),
)


def _fc_body(aggh_ref, den_ref, wfc_ref, bfc_ref, out_ref):
    acc = jnp.zeros((BN, 16), jnp.float32) + bfc_ref[...]
    for m in range(M):
        for h in range(H):
            c, off = h // 2, (h % 2) * DH
            v = aggh_ref[c, m, :, off:off + DH] / (den_ref[m, :, h:h + 1] + 1e-9)
            ev = _elu(v)
            w = wfc_ref[(m * H + h) * DH:(m * H + h + 1) * DH, :]
            acc = acc + jnp.dot(ev, w, preferred_element_type=jnp.float32)
    out_ref[...] = acc


_fc = pl.pallas_call(
    _fc_body,
    grid=(NB,),
    in_specs=[
        pl.BlockSpec((NC, M, BN, FH), lambda i: (0, 0, i, 0)),
        pl.BlockSpec((M, BN, H), lambda i: (0, i, 0)),
        pl.BlockSpec((M * F, 16), lambda i: (0, 0)),
        pl.BlockSpec((1, 16), lambda i: (0, 0)),
    ],
    out_specs=pl.BlockSpec((BN, 16), lambda i: (i, 0)),
    out_shape=jax.ShapeDtypeStruct((N, 16), jnp.float32),
)


def _make_smat(a_src, a_dst):
    sm = jnp.zeros((M, F, 16), jnp.float32)
    for h in range(H):
        sm = sm.at[:, h * DH:(h + 1) * DH, h].set(a_src[:, h, :])
        sm = sm.at[:, h * DH:(h + 1) * DH, H + h].set(a_dst[:, h, :])
    return sm


def _layer(xin, Wr, smat, gsrc_f, gdst_f, gsrc, dstl, z128, z4, dense_fn):
    hph, stab = dense_fn(xin, Wr, smat)
    ex = _edge_ex(stab.reshape(M * N, 16), gsrc_f, gdst_f)
    aggh, den = _agg(hph.reshape(NC * M * N, FH), gsrc, dstl,
                     ex.reshape(M, E, H), z128, z4)
    return aggh, den


def kernel(x, edge_index, W0, a_src0, a_dst0, attn_q, W1, a_src1, a_dst1,
           Wfc, bfc):
    Wr0 = jnp.transpose(W0, (0, 2, 1, 3)).reshape(M, DIN, F)
    Wr1 = jnp.transpose(W1, (0, 2, 1, 3)).reshape(M, DH, F)
    smat0 = _make_smat(a_src0, a_dst0)
    smat1 = _make_smat(a_src1, a_dst1)

    offs = (jnp.arange(M, dtype=jnp.int32) * N)[:, None]
    gsrc = edge_index[:, 0, :] + offs
    gdst = edge_index[:, 1, :] + offs
    gsrc_f = gsrc.reshape(-1)
    gdst_f = gdst.reshape(-1)
    dstl = edge_index[:, 1, :]

    z128 = jnp.zeros((RPT, FH), jnp.float32)
    z4 = jnp.zeros((RPT, H), jnp.float32)

    aggh0, den0 = _layer(x, Wr0, smat0, gsrc_f, gdst_f, gsrc, dstl,
                         z128, z4, _dense0)
    hmid = _mc(aggh0, den0, attn_q.reshape(1, DH))
    aggh1, den1 = _layer(hmid, Wr1, smat1, gsrc_f, gdst_f, gsrc, dstl,
                         z128, z4, _dense1)
    return _fc(aggh1, den1, Wfc, bfc.reshape(1, 16))


# speedup vs baseline: 15.1888x; 15.1888x over previous
import functools

import jax
import jax.numpy as jnp
from jax import lax
from jax.experimental import pallas as pl
from jax.experimental.pallas import tpu as pltpu
from jax.experimental.pallas import tpu_sc as plsc

N = 10000
E = 320000
M = 3
H = 4
DH = 64
DIN = 128
F = H * DH
FH = F // 2

NC = 2
NS = 16
NW = NC * NS
RPT = N // NS

TW = M * E // NW
KE = 120
NCHUNK_E = TW // KE

EPT = E // NS
KA = 80
NCHUNK_A = EPT // KA

BN = 2000
NB = N // BN

_mesh = plsc.VectorSubcoreMesh(
    core_axis_name="c", subcore_axis_name="s", num_cores=NC, num_subcores=NS)


_sc_params = pltpu.CompilerParams(needs_layout_passes=False,
                                  use_tc_tiling_on_sc=False)


@functools.partial(
    pl.kernel,
    out_type=jax.ShapeDtypeStruct((M * E * 4,), jnp.float32),
    mesh=_mesh,
    compiler_params=_sc_params,
    scratch_types=[
        pltpu.VMEM((KE,), jnp.int32),
        pltpu.VMEM((KE,), jnp.int32),
        pltpu.VMEM((KE, 16), jnp.float32),
        pltpu.VMEM((KE, 16), jnp.float32),
        pltpu.VMEM((KE * 4,), jnp.float32),
        pltpu.SemaphoreType.DMA,
        pltpu.SemaphoreType.DMA,
    ],
)
def _edge_ex(stab_hbm, gsrc_hbm, gdst_hbm, ex_hbm,
             sidx, didx, sv, dv, exv, sem1, sem2):
    wid = lax.axis_index("s") * NC + lax.axis_index("c")
    base0 = wid * TW

    def chunk(i, carry):
        b = base0 + i * KE
        pltpu.sync_copy(gsrc_hbm.at[pl.ds(b, KE)], sidx)
        pltpu.sync_copy(gdst_hbm.at[pl.ds(b, KE)], didx)
        cp1 = pltpu.async_copy(stab_hbm.at[sidx], sv, sem1)
        cp2 = pltpu.async_copy(stab_hbm.at[didx], dv, sem2)
        cp1.wait()
        cp2.wait()

        def step(t, c2):
            j = t * 16 + lax.iota(jnp.int32, 16)
            e = lax.shift_right_logical(j, 2)
            hh = lax.bitwise_and(j, 3)
            a = plsc.load_gather(sv, [e, hh])
            bb = plsc.load_gather(dv, [e, hh + 4])
            s = a + bb
            s = jnp.maximum(s, 0.2 * s)
            exv[pl.ds(t * 16, 16)] = jnp.exp(s)
            return c2

        lax.fori_loop(0, KE * 4 // 16, step, 0)
        pltpu.sync_copy(exv, ex_hbm.at[pl.ds(b * 4, KE * 4)])
        return carry

    lax.fori_loop(0, NCHUNK_E, chunk, 0)


@functools.partial(
    pl.kernel,
    out_type=(
        jax.ShapeDtypeStruct((NC, M, N, FH), jnp.float32),
        jax.ShapeDtypeStruct((M, N, H), jnp.float32),
    ),
    mesh=_mesh,
    compiler_params=_sc_params,
    scratch_types=[
        pltpu.VMEM((160, FH), jnp.float32),
        pltpu.VMEM((160, H), jnp.float32),
        pltpu.VMEM((KA, FH), jnp.float32),
        pltpu.VMEM((KA,), jnp.int32),
        pltpu.VMEM((KA,), jnp.int32),
        pltpu.VMEM((KA,), jnp.int32),
        pltpu.VMEM((KA, H), jnp.float32),
        pltpu.VMEM_SHARED((N, FH), jnp.float32),
        pltpu.VMEM_SHARED((N, H), jnp.float32),
        pltpu.SemaphoreType.DMA,
    ],
)
def _agg(hp_cm, gsrc_hbm, dst_hbm, ex_hbm, z128_hbm, z4_hbm,
         agg_out, den_out,
         zbuf, zd, rows, sidx, sidx2, didx, exb, acc, dacc, sem):
    cid = lax.axis_index("c")
    sid = lax.axis_index("s")
    r0 = sid * RPT
    coff = cid * (M * N)
    c2 = cid * 2

    zch = ((0, 160), (160, 160), (320, 160), (480, RPT - 480))

    for m in range(M):
        pltpu.sync_copy(z128_hbm, zbuf)
        for off, cnt in zch:
            pltpu.sync_copy(zbuf.at[pl.ds(0, cnt)],
                            acc.at[pl.ds(r0 + off, cnt)])

        @pl.when(cid == 0)
        def _():
            pltpu.sync_copy(z4_hbm, zd)
            for off, cnt in zch:
                pltpu.sync_copy(zd.at[pl.ds(0, cnt)],
                                dacc.at[pl.ds(r0 + off, cnt)])

        plsc.subcore_barrier()

        def chunk(i, carry):
            b = sid * EPT + i * KA
            pltpu.sync_copy(gsrc_hbm.at[m, pl.ds(b, KA)], sidx)
            pltpu.sync_copy(dst_hbm.at[m, pl.ds(b, KA)], didx)
            pltpu.sync_copy(ex_hbm.at[m, pl.ds(b, KA)], exb)
            for t in range(KA // 16):
                sidx2[pl.ds(t * 16, 16)] = sidx[pl.ds(t * 16, 16)] + coff
            pltpu.async_copy(hp_cm.at[sidx2], rows, sem).wait()

            def scale(e2, cc):
                ev = jnp.full((16,), e2, jnp.int32)
                s0 = plsc.load_gather(exb, [ev, jnp.full((16,), c2, jnp.int32)])
                s1 = plsc.load_gather(exb, [ev, jnp.full((16,), c2 + 1, jnp.int32)])
                for j in range(4):
                    rows[e2, pl.ds(j * 16, 16)] = rows[e2, pl.ds(j * 16, 16)] * s0
                for j in range(4, 8):
                    rows[e2, pl.ds(j * 16, 16)] = rows[e2, pl.ds(j * 16, 16)] * s1
                return cc

            lax.fori_loop(0, KA, scale, 0)
            pltpu.sync_copy(rows, acc.at[didx], add=True)

            @pl.when(cid == 0)
            def _():
                pltpu.sync_copy(exb, dacc.at[didx], add=True)

            return carry

        lax.fori_loop(0, NCHUNK_A, chunk, 0)
        plsc.subcore_barrier()

        for off, cnt in zch:
            pltpu.sync_copy(acc.at[pl.ds(r0 + off, cnt)],
                            zbuf.at[pl.ds(0, cnt)])
            pltpu.sync_copy(zbuf.at[pl.ds(0, cnt)],
                            agg_out.at[cid, m, pl.ds(r0 + off, cnt), :])

        @pl.when(cid == 0)
        def _():
            for off, cnt in zch:
                pltpu.sync_copy(dacc.at[pl.ds(r0 + off, cnt)],
                                zd.at[pl.ds(0, cnt)])
                pltpu.sync_copy(zd.at[pl.ds(0, cnt)],
                                den_out.at[m, pl.ds(r0 + off, cnt), :])

        plsc.subcore_barrier()


def _dense_body(x_ref, w_ref, smat_ref, hph_ref, stab_ref):
    xb = x_ref[...]
    hp = jnp.dot(xb, w_ref[0], preferred_element_type=jnp.float32)
    hph_ref[0, 0] = hp[:, :FH]
    hph_ref[1, 0] = hp[:, FH:]
    stab_ref[0] = jnp.dot(hp, smat_ref[0], preferred_element_type=jnp.float32)


def _make_dense(din):
    return pl.pallas_call(
        _dense_body,
        grid=(M, NB),
        in_specs=[
            pl.BlockSpec((BN, din), lambda m, i: (i, 0)),
            pl.BlockSpec((1, din, F), lambda m, i: (m, 0, 0)),
            pl.BlockSpec((1, F, 16), lambda m, i: (m, 0, 0)),
        ],
        out_specs=[
            pl.BlockSpec((NC, 1, BN, FH), lambda m, i: (0, m, i, 0)),
            pl.BlockSpec((1, BN, 16), lambda m, i: (m, i, 0)),
        ],
        out_shape=[
            jax.ShapeDtypeStruct((NC, M, N, FH), jnp.float32),
            jax.ShapeDtypeStruct((M, N, 16), jnp.float32),
        ],
    )


_dense0 = _make_dense(DIN)
_dense1 = _make_dense(DH)


def _elu(v):
    return jnp.where(v > 0, v, jnp.exp(v) - 1.0)


def _mc_body(aggh_ref, den_ref, q_ref, h_ref):
    zs = []
    for m in range(M):
        acc = None
        for h in range(H):
            c, off = h // 2, (h % 2) * DH
            v = aggh_ref[c, m, :, off:off + DH] / (den_ref[m, :, h:h + 1] + 1e-9)
            ev = _elu(v)
            acc = ev if acc is None else acc + ev
        zs.append(acc * (1.0 / H))
    q = q_ref[...]
    ss = [jnp.sum(jnp.tanh(z) * q, axis=1, keepdims=True) for z in zs]
    smax = jnp.maximum(jnp.maximum(ss[0], ss[1]), ss[2])
    es = [jnp.exp(s - smax) for s in ss]
    tot = es[0] + es[1] + es[2]
    hsum = sum((e / tot) * z for e, z in zip(es, zs))
    h_ref[...] = jnp.maximum(hsum, 0.0)


_mc = pl.pallas_call(
    _mc_body,
    grid=(NB,),
    in_specs=[
        pl.BlockSpec((NC, M, BN, FH), lambda i: (0, 0, i, 0)),
        pl.BlockSpec((M, BN, H), lambda i: (0, i, 0)),
        pl.BlockSpec((1, DH), lambda i: (0, 0)),
    ],
    out_specs=pl.BlockSpec((BN, DH), lambda i: (i, 0)),
    out_shape=jax.ShapeDtypeStruct((N, DH), jnp.float32),
)


def _fc_body(aggh_ref, den_ref, wfc_ref, bfc_ref, out_ref):
    acc = jnp.zeros((BN, 16), jnp.float32) + bfc_ref[...]
    for m in range(M):
        for h in range(H):
            c, off = h // 2, (h % 2) * DH
            v = aggh_ref[c, m, :, off:off + DH] / (den_ref[m, :, h:h + 1] + 1e-9)
            ev = _elu(v)
            w = wfc_ref[(m * H + h) * DH:(m * H + h + 1) * DH, :]
            acc = acc + jnp.dot(ev, w, preferred_element_type=jnp.float32)
    out_ref[...] = acc


_fc = pl.pallas_call(
    _fc_body,
    grid=(NB,),
    in_specs=[
        pl.BlockSpec((NC, M, BN, FH), lambda i: (0, 0, i, 0)),
        pl.BlockSpec((M, BN, H), lambda i: (0, i, 0)),
        pl.BlockSpec((M * F, 16), lambda i: (0, 0)),
        pl.BlockSpec((1, 16), lambda i: (0, 0)),
    ],
    out_specs=pl.BlockSpec((BN, 16), lambda i: (i, 0)),
    out_shape=jax.ShapeDtypeStruct((N, 16), jnp.float32),
)


def _make_smat(a_src, a_dst):
    sm = jnp.zeros((M, F, 16), jnp.float32)
    for h in range(H):
        sm = sm.at[:, h * DH:(h + 1) * DH, h].set(a_src[:, h, :])
        sm = sm.at[:, h * DH:(h + 1) * DH, H + h].set(a_dst[:, h, :])
    return sm


def _layer(xin, Wr, smat, gsrc_f, gdst_f, gsrc, dstl, z128, z4, dense_fn):
    hph, stab = dense_fn(xin, Wr, smat)
    ex = _edge_ex(stab.reshape(M * N, 16), gsrc_f, gdst_f)
    aggh, den = _agg(hph.reshape(NC * M * N, FH), gsrc, dstl,
                     ex.reshape(M, E, H), z128, z4)
    return aggh, den


def kernel(x, edge_index, W0, a_src0, a_dst0, attn_q, W1, a_src1, a_dst1,
           Wfc, bfc):
    Wr0 = jnp.transpose(W0, (0, 2, 1, 3)).reshape(M, DIN, F)
    Wr1 = jnp.transpose(W1, (0, 2, 1, 3)).reshape(M, DH, F)
    smat0 = _make_smat(a_src0, a_dst0)
    smat1 = _make_smat(a_src1, a_dst1)

    offs = (jnp.arange(M, dtype=jnp.int32) * N)[:, None]
    gsrc = edge_index[:, 0, :] + offs
    gdst = edge_index[:, 1, :] + offs
    gsrc_f = gsrc.reshape(-1)
    gdst_f = gdst.reshape(-1)
    dstl = edge_index[:, 1, :]

    z128 = jnp.zeros((160, FH), jnp.float32)
    z4 = jnp.zeros((160, H), jnp.float32)

    aggh0, den0 = _layer(x, Wr0, smat0, gsrc_f, gdst_f, gsrc, dstl,
                         z128, z4, _dense0)
    hmid = _mc(aggh0, den0, attn_q.reshape(1, DH))
    aggh1, den1 = _layer(hmid, Wr1, smat1, gsrc_f, gdst_f, gsrc, dstl,
                         z128, z4, _dense1)
    return _fc(aggh1, den1, Wfc, bfc.reshape(1, 16))


# speedup vs baseline: 20.0593x; 1.3207x over previous
import functools

import jax
import jax.numpy as jnp
from jax import lax
from jax.experimental import pallas as pl
from jax.experimental.pallas import tpu as pltpu
from jax.experimental.pallas import tpu_sc as plsc

N = 10000
E = 320000
M = 3
H = 4
DH = 64
DIN = 128
F = H * DH
FH = F // 2

NC = 2
NS = 16
NW = NC * NS
RPT = N // NS

TW = M * E // NW
KE = 120
NCHUNK_E = TW // KE

EPT = E // NS
KA = 80
NCHUNK_A = EPT // KA

BN = 2000
NB = N // BN

_mesh = plsc.VectorSubcoreMesh(
    core_axis_name="c", subcore_axis_name="s", num_cores=NC, num_subcores=NS)


_sc_params = pltpu.CompilerParams(needs_layout_passes=False,
                                  use_tc_tiling_on_sc=False)


@functools.partial(
    pl.kernel,
    out_type=jax.ShapeDtypeStruct((M * E * 4,), jnp.float32),
    mesh=_mesh,
    compiler_params=_sc_params,
    scratch_types=[
        pltpu.VMEM((KE,), jnp.int32),
        pltpu.VMEM((KE,), jnp.int32),
        pltpu.VMEM((KE, 16), jnp.float32),
        pltpu.VMEM((KE, 16), jnp.float32),
        pltpu.VMEM((KE * 4,), jnp.float32),
        pltpu.SemaphoreType.DMA,
        pltpu.SemaphoreType.DMA,
    ],
)
def _edge_ex(stab_hbm, gsrc_hbm, gdst_hbm, ex_hbm,
             sidx, didx, sv, dv, exv, sem1, sem2):
    wid = lax.axis_index("s") * NC + lax.axis_index("c")
    base0 = wid * TW

    def chunk(i, carry):
        b = base0 + i * KE
        pltpu.sync_copy(gsrc_hbm.at[pl.ds(b, KE)], sidx)
        pltpu.sync_copy(gdst_hbm.at[pl.ds(b, KE)], didx)
        cp1 = pltpu.async_copy(stab_hbm.at[sidx], sv, sem1)
        cp2 = pltpu.async_copy(stab_hbm.at[didx], dv, sem2)
        cp1.wait()
        cp2.wait()

        def step(t, c2):
            j = t * 16 + lax.iota(jnp.int32, 16)
            e = lax.shift_right_logical(j, 2)
            hh = lax.bitwise_and(j, 3)
            a = plsc.load_gather(sv, [e, hh])
            bb = plsc.load_gather(dv, [e, hh + 4])
            s = a + bb
            s = jnp.maximum(s, 0.2 * s)
            exv[pl.ds(t * 16, 16)] = jnp.exp(s)
            return c2

        lax.fori_loop(0, KE * 4 // 16, step, 0)
        pltpu.sync_copy(exv, ex_hbm.at[pl.ds(b * 4, KE * 4)])
        return carry

    lax.fori_loop(0, NCHUNK_E, chunk, 0)


@functools.partial(
    pl.kernel,
    out_type=(
        jax.ShapeDtypeStruct((NC, M, N, FH), jnp.float32),
        jax.ShapeDtypeStruct((M, N, H), jnp.float32),
    ),
    mesh=_mesh,
    compiler_params=_sc_params,
    scratch_types=(
        [pltpu.VMEM((40, FH), jnp.float32),
         pltpu.VMEM((160, H), jnp.float32)]
        + [pltpu.VMEM((KA, FH), jnp.float32)] * 3
        + [pltpu.VMEM((KA,), jnp.int32)] * 3
        + [pltpu.VMEM((KA,), jnp.int32)] * 3
        + [pltpu.VMEM((KA, H), jnp.float32)] * 3
        + [pltpu.VMEM_SHARED((N, FH), jnp.float32),
           pltpu.VMEM_SHARED((N, H), jnp.float32)]
        + [pltpu.SemaphoreType.DMA] * 9
    ),
)
def _agg(hp_cm, gsrc_hbm, dst_hbm, ex_hbm, z128_hbm, z4_hbm,
         agg_out, den_out, *scr):
    zbuf, zd = scr[0], scr[1]
    rows = scr[2:5]
    sidx = scr[5:8]
    didx = scr[8:11]
    exb = scr[11:14]
    acc, dacc = scr[14], scr[15]
    semg = scr[16:19]
    sems = scr[19:22]
    semd = scr[22:25]

    cid = lax.axis_index("c")
    sid = lax.axis_index("s")
    r0 = sid * RPT
    coff = cid * (M * N)
    c2 = cid * 2

    zchd = ((0, 160), (160, 160), (320, 160), (480, RPT - 480))

    def stage(k, m, ci):
        b = sid * EPT + ci * KA
        pltpu.sync_copy(gsrc_hbm.at[m, pl.ds(b, KA)], sidx[k])
        pltpu.sync_copy(dst_hbm.at[m, pl.ds(b, KA)], didx[k])
        pltpu.sync_copy(ex_hbm.at[m, pl.ds(b, KA)], exb[k])
        for t in range(KA // 16):
            sidx[k][pl.ds(t * 16, 16)] = sidx[k][pl.ds(t * 16, 16)] + coff
        pltpu.async_copy(hp_cm.at[sidx[k]], rows[k], semg[k])

    def wait_gather(k):
        pltpu.make_async_copy(hp_cm.at[sidx[k]], rows[k], semg[k]).wait()

    def scale(k):
        def body(e2, cc):
            ev = jnp.full((16,), e2, jnp.int32)
            s0 = plsc.load_gather(exb[k], [ev, jnp.full((16,), c2, jnp.int32)])
            s1 = plsc.load_gather(exb[k],
                                  [ev, jnp.full((16,), c2 + 1, jnp.int32)])
            for j in range(4):
                rows[k][e2, pl.ds(j * 16, 16)] = (
                    rows[k][e2, pl.ds(j * 16, 16)] * s0)
            for j in range(4, 8):
                rows[k][e2, pl.ds(j * 16, 16)] = (
                    rows[k][e2, pl.ds(j * 16, 16)] * s1)
            return cc

        lax.fori_loop(0, KA, body, 0)

    def start_scatter(k):
        pltpu.async_copy(rows[k], acc.at[didx[k]], sems[k], add=True)

        @pl.when(cid == 0)
        def _():
            pltpu.async_copy(exb[k], dacc.at[didx[k]], semd[k], add=True)

    def wait_scatter(k):
        pltpu.make_async_copy(rows[k], acc.at[didx[k]], sems[k]).wait()

        @pl.when(cid == 0)
        def _():
            pltpu.make_async_copy(exb[k], dacc.at[didx[k]], semd[k]).wait()

    for m in range(M):
        pltpu.sync_copy(z128_hbm, zbuf)
        for kk in range(16):
            off = kk * 40
            cnt = 40 if off + 40 <= RPT else RPT - off
            pltpu.sync_copy(zbuf.at[pl.ds(0, cnt)],
                            acc.at[pl.ds(r0 + off, cnt)])

        @pl.when(cid == 0)
        def _():
            pltpu.sync_copy(z4_hbm, zd)
            for off, cnt in zchd:
                pltpu.sync_copy(zd.at[pl.ds(0, cnt)],
                                dacc.at[pl.ds(r0 + off, cnt)])

        plsc.subcore_barrier()

        stage(0, m, 0)

        def sstep(s, cc):
            for b in range(3):
                i = s * 3 + b
                kn = (b + 1) % 3

                @pl.when(i >= 2)
                def _():
                    wait_scatter(kn)

                stage(kn, m, i + 1)
                wait_gather(b)
                scale(b)
                start_scatter(b)
            return cc

        lax.fori_loop(0, (NCHUNK_A - 1) // 3, sstep, 0)
        wait_scatter(1)
        wait_gather(0)
        scale(0)
        start_scatter(0)
        wait_scatter(2)
        wait_scatter(0)
        plsc.subcore_barrier()

        for kk in range(16):
            off = kk * 40
            cnt = 40 if off + 40 <= RPT else RPT - off
            pltpu.sync_copy(acc.at[pl.ds(r0 + off, cnt)],
                            zbuf.at[pl.ds(0, cnt)])
            pltpu.sync_copy(zbuf.at[pl.ds(0, cnt)],
                            agg_out.at[cid, m, pl.ds(r0 + off, cnt), :])

        @pl.when(cid == 0)
        def _():
            for off, cnt in zchd:
                pltpu.sync_copy(dacc.at[pl.ds(r0 + off, cnt)],
                                zd.at[pl.ds(0, cnt)])
                pltpu.sync_copy(zd.at[pl.ds(0, cnt)],
                                den_out.at[m, pl.ds(r0 + off, cnt), :])

        plsc.subcore_barrier()


def _dense_body(x_ref, w_ref, smat_ref, hph_ref, stab_ref):
    xb = x_ref[...]
    hp = jnp.dot(xb, w_ref[0], preferred_element_type=jnp.float32)
    hph_ref[0, 0] = hp[:, :FH]
    hph_ref[1, 0] = hp[:, FH:]
    stab_ref[0] = jnp.dot(hp, smat_ref[0], preferred_element_type=jnp.float32)


def _make_dense(din):
    return pl.pallas_call(
        _dense_body,
        grid=(M, NB),
        in_specs=[
            pl.BlockSpec((BN, din), lambda m, i: (i, 0)),
            pl.BlockSpec((1, din, F), lambda m, i: (m, 0, 0)),
            pl.BlockSpec((1, F, 16), lambda m, i: (m, 0, 0)),
        ],
        out_specs=[
            pl.BlockSpec((NC, 1, BN, FH), lambda m, i: (0, m, i, 0)),
            pl.BlockSpec((1, BN, 16), lambda m, i: (m, i, 0)),
        ],
        out_shape=[
            jax.ShapeDtypeStruct((NC, M, N, FH), jnp.float32),
            jax.ShapeDtypeStruct((M, N, 16), jnp.float32),
        ],
    )


_dense0 = _make_dense(DIN)
_dense1 = _make_dense(DH)


def _elu(v):
    return jnp.where(v > 0, v, jnp.exp(v) - 1.0)


def _mc_body(aggh_ref, den_ref, q_ref, h_ref):
    zs = []
    for m in range(M):
        acc = None
        for h in range(H):
            c, off = h // 2, (h % 2) * DH
            v = aggh_ref[c, m, :, off:off + DH] / (den_ref[m, :, h:h + 1] + 1e-9)
            ev = _elu(v)
            acc = ev if acc is None else acc + ev
        zs.append(acc * (1.0 / H))
    q = q_ref[...]
    ss = [jnp.sum(jnp.tanh(z) * q, axis=1, keepdims=True) for z in zs]
    smax = jnp.maximum(jnp.maximum(ss[0], ss[1]), ss[2])
    es = [jnp.exp(s - smax) for s in ss]
    tot = es[0] + es[1] + es[2]
    hsum = sum((e / tot) * z for e, z in zip(es, zs))
    h_ref[...] = jnp.maximum(hsum, 0.0)


_mc = pl.pallas_call(
    _mc_body,
    grid=(NB,),
    in_specs=[
        pl.BlockSpec((NC, M, BN, FH), lambda i: (0, 0, i, 0)),
        pl.BlockSpec((M, BN, H), lambda i: (0, i, 0)),
        pl.BlockSpec((1, DH), lambda i: (0, 0)),
    ],
    out_specs=pl.BlockSpec((BN, DH), lambda i: (i, 0)),
    out_shape=jax.ShapeDtypeStruct((N, DH), jnp.float32),
)


def _fc_body(aggh_ref, den_ref, wfc_ref, bfc_ref, out_ref):
    acc = jnp.zeros((BN, 16), jnp.float32) + bfc_ref[...]
    for m in range(M):
        for h in range(H):
            c, off = h // 2, (h % 2) * DH
            v = aggh_ref[c, m, :, off:off + DH] / (den_ref[m, :, h:h + 1] + 1e-9)
            ev = _elu(v)
            w = wfc_ref[(m * H + h) * DH:(m * H + h + 1) * DH, :]
            acc = acc + jnp.dot(ev, w, preferred_element_type=jnp.float32)
    out_ref[...] = acc


_fc = pl.pallas_call(
    _fc_body,
    grid=(NB,),
    in_specs=[
        pl.BlockSpec((NC, M, BN, FH), lambda i: (0, 0, i, 0)),
        pl.BlockSpec((M, BN, H), lambda i: (0, i, 0)),
        pl.BlockSpec((M * F, 16), lambda i: (0, 0)),
        pl.BlockSpec((1, 16), lambda i: (0, 0)),
    ],
    out_specs=pl.BlockSpec((BN, 16), lambda i: (i, 0)),
    out_shape=jax.ShapeDtypeStruct((N, 16), jnp.float32),
)


def _make_smat(a_src, a_dst):
    sm = jnp.zeros((M, F, 16), jnp.float32)
    for h in range(H):
        sm = sm.at[:, h * DH:(h + 1) * DH, h].set(a_src[:, h, :])
        sm = sm.at[:, h * DH:(h + 1) * DH, H + h].set(a_dst[:, h, :])
    return sm


def _layer(xin, Wr, smat, gsrc_f, gdst_f, gsrc, dstl, z128, z4, dense_fn):
    hph, stab = dense_fn(xin, Wr, smat)
    ex = _edge_ex(stab.reshape(M * N, 16), gsrc_f, gdst_f)
    aggh, den = _agg(hph.reshape(NC * M * N, FH), gsrc, dstl,
                     ex.reshape(M, E, H), z128, z4)
    return aggh, den


def kernel(x, edge_index, W0, a_src0, a_dst0, attn_q, W1, a_src1, a_dst1,
           Wfc, bfc):
    Wr0 = jnp.transpose(W0, (0, 2, 1, 3)).reshape(M, DIN, F)
    Wr1 = jnp.transpose(W1, (0, 2, 1, 3)).reshape(M, DH, F)
    smat0 = _make_smat(a_src0, a_dst0)
    smat1 = _make_smat(a_src1, a_dst1)

    offs = (jnp.arange(M, dtype=jnp.int32) * N)[:, None]
    gsrc = edge_index[:, 0, :] + offs
    gdst = edge_index[:, 1, :] + offs
    gsrc_f = gsrc.reshape(-1)
    gdst_f = gdst.reshape(-1)
    dstl = edge_index[:, 1, :]

    z128 = jnp.zeros((40, FH), jnp.float32)
    z4 = jnp.zeros((160, H), jnp.float32)

    aggh0, den0 = _layer(x, Wr0, smat0, gsrc_f, gdst_f, gsrc, dstl,
                         z128, z4, _dense0)
    hmid = _mc(aggh0, den0, attn_q.reshape(1, DH))
    aggh1, den1 = _layer(hmid, Wr1, smat1, gsrc_f, gdst_f, gsrc, dstl,
                         z128, z4, _dense1)
    return _fc(aggh1, den1, Wfc, bfc.reshape(1, 16))


# speedup vs baseline: 58.3810x; 2.9104x over previous
import functools

import jax
import jax.numpy as jnp
from jax import lax
from jax.experimental import pallas as pl
from jax.experimental.pallas import tpu as pltpu
from jax.experimental.pallas import tpu_sc as plsc

N = 10000
E = 320000
M = 3
H = 4
DH = 64
DIN = 128
F = H * DH
FH = F // 2

NC = 2
NS = 16
RPT = N // NS

EPT = E // NS
KA = 40
NCH = EPT // KA
NSET = 5

BN = 2000
NB = N // BN

_mesh = plsc.VectorSubcoreMesh(
    core_axis_name="c", subcore_axis_name="s", num_cores=NC, num_subcores=NS)

_sc_params = pltpu.CompilerParams(needs_layout_passes=False,
                                  use_tc_tiling_on_sc=False)


@functools.partial(
    pl.kernel,
    out_type=(
        jax.ShapeDtypeStruct((NC, M, N, FH), jnp.float32),
        jax.ShapeDtypeStruct((M, N, H), jnp.float32),
    ),
    mesh=_mesh,
    compiler_params=_sc_params,
    scratch_types=(
        [pltpu.VMEM((40, FH), jnp.float32),
         pltpu.VMEM((160, H), jnp.float32)]
        + [pltpu.VMEM((KA, FH), jnp.float32)] * NSET
        + [pltpu.VMEM((KA,), jnp.int32)] * NSET
        + [pltpu.VMEM((KA,), jnp.int32)] * NSET
        + [pltpu.VMEM((KA,), jnp.int32)] * NSET
        + [pltpu.VMEM((KA,), jnp.int32)] * NSET
        + [pltpu.VMEM((KA, 16), jnp.float32)] * NSET
        + [pltpu.VMEM((KA, 16), jnp.float32)] * NSET
        + [pltpu.VMEM((KA, H), jnp.float32)] * NSET
        + [pltpu.VMEM_SHARED((N, FH), jnp.float32),
           pltpu.VMEM_SHARED((N, H), jnp.float32)]
        + [pltpu.SemaphoreType.DMA] * (5 * NSET)
    ),
)
def _agg(hp_cm, stab_hbm, gsrc_hbm, gsrc2_hbm, dst_hbm, gdst_hbm,
         z128_hbm, z4_hbm, agg_out, den_out, *scr):
    zbuf, zd = scr[0], scr[1]
    o = 2
    rows = scr[o:o + NSET]; o += NSET
    sidxg = scr[o:o + NSET]; o += NSET
    sidx2 = scr[o:o + NSET]; o += NSET
    didx = scr[o:o + NSET]; o += NSET
    gdix = scr[o:o + NSET]; o += NSET
    sv = scr[o:o + NSET]; o += NSET
    dv = scr[o:o + NSET]; o += NSET
    exb = scr[o:o + NSET]; o += NSET
    acc, dacc = scr[o], scr[o + 1]; o += 2
    semt = scr[o:o + NSET]; o += NSET
    semv = scr[o:o + NSET]; o += NSET
    semg = scr[o:o + NSET]; o += NSET
    sems = scr[o:o + NSET]; o += NSET
    semd = scr[o:o + NSET]

    cid = lax.axis_index("c")
    sid = lax.axis_index("s")
    r0 = sid * RPT
    c2 = cid * 2

    zchd = ((0, 160), (160, 160), (320, 160), (480, RPT - 480))

    def stage_list(k, m, ci):
        b = sid * EPT + ci * KA
        return (
            (gsrc_hbm.at[m, pl.ds(b, KA)], sidxg[k]),
            (gsrc2_hbm.at[cid, m, pl.ds(b, KA)], sidx2[k]),
            (dst_hbm.at[m, pl.ds(b, KA)], didx[k]),
            (gdst_hbm.at[m, pl.ds(b, KA)], gdix[k]),
        )

    def fire_stage(k, m, ci):
        for s, d in stage_list(k, m, ci):
            pltpu.async_copy(s, d, semt[k])

    def wait_stage(k, m, ci):
        for s, d in stage_list(k, m, ci):
            pltpu.make_async_copy(s, d, semt[k]).wait()

    def fire_gathers(k):
        pltpu.async_copy(stab_hbm.at[sidxg[k]], sv[k], semv[k])
        pltpu.async_copy(stab_hbm.at[gdix[k]], dv[k], semv[k])
        pltpu.async_copy(hp_cm.at[sidx2[k]], rows[k], semg[k])

    def wait_svdv(k):
        pltpu.make_async_copy(stab_hbm.at[sidxg[k]], sv[k], semv[k]).wait()
        pltpu.make_async_copy(stab_hbm.at[gdix[k]], dv[k], semv[k]).wait()

    def wait_hp(k):
        pltpu.make_async_copy(hp_cm.at[sidx2[k]], rows[k], semg[k]).wait()

    def excomp(k):
        def body(t, cc):
            j = t * 16 + lax.iota(jnp.int32, 16)
            e = lax.shift_right_logical(j, 2)
            hh = lax.bitwise_and(j, 3)
            a = plsc.load_gather(sv[k], [e, hh])
            bb = plsc.load_gather(dv[k], [e, hh + 4])
            s = a + bb
            s = jnp.maximum(s, 0.2 * s)
            plsc.store_scatter(exb[k], [e, hh], jnp.exp(s))
            return cc

        lax.fori_loop(0, KA * H // 16, body, 0)

    def scale(k):
        def body(e2, cc):
            ev = jnp.full((16,), e2, jnp.int32)
            s0 = plsc.load_gather(exb[k], [ev, jnp.full((16,), c2, jnp.int32)])
            s1 = plsc.load_gather(exb[k],
                                  [ev, jnp.full((16,), c2 + 1, jnp.int32)])
            for j in range(4):
                rows[k][e2, pl.ds(j * 16, 16)] = (
                    rows[k][e2, pl.ds(j * 16, 16)] * s0)
            for j in range(4, 8):
                rows[k][e2, pl.ds(j * 16, 16)] = (
                    rows[k][e2, pl.ds(j * 16, 16)] * s1)
            return cc

        lax.fori_loop(0, KA, body, 0)

    def fire_scatter(k):
        pltpu.async_copy(rows[k], acc.at[didx[k]], sems[k], add=True)

        @pl.when(cid == 0)
        def _():
            pltpu.async_copy(exb[k], dacc.at[didx[k]], semd[k], add=True)

    def wait_scatter(k):
        pltpu.make_async_copy(rows[k], acc.at[didx[k]], sems[k]).wait()

        @pl.when(cid == 0)
        def _():
            pltpu.make_async_copy(exb[k], dacc.at[didx[k]], semd[k]).wait()

    for m in range(M):
        pltpu.sync_copy(z128_hbm, zbuf)
        for kk in range(16):
            off = kk * 40
            cnt = 40 if off + 40 <= RPT else RPT - off
            pltpu.sync_copy(zbuf.at[pl.ds(0, cnt)],
                            acc.at[pl.ds(r0 + off, cnt)])

        @pl.when(cid == 0)
        def _():
            pltpu.sync_copy(z4_hbm, zd)
            for off, cnt in zchd:
                pltpu.sync_copy(zd.at[pl.ds(0, cnt)],
                                dacc.at[pl.ds(r0 + off, cnt)])

        plsc.subcore_barrier()

        fire_stage(0, m, 0)
        fire_stage(1, m, 1)
        fire_stage(2, m, 2)
        wait_stage(0, m, 0)
        fire_gathers(0)

        def sstep(s, cc):
            for b in range(NSET):
                i = s * NSET + b

                @pl.when(i >= 2)
                def _():
                    wait_scatter((b + 3) % NSET)

                @pl.when(i < NCH - 3)
                def _():
                    fire_stage((b + 3) % NSET, m, i + 3)

                @pl.when(i < NCH - 1)
                def _():
                    wait_stage((b + 1) % NSET, m, i + 1)
                    fire_gathers((b + 1) % NSET)

                wait_svdv(b)
                excomp(b)
                wait_hp(b)
                scale(b)
                fire_scatter(b)
            return cc

        lax.fori_loop(0, NCH // NSET, sstep, 0)
        wait_scatter((NCH - 2) % NSET)
        wait_scatter((NCH - 1) % NSET)
        plsc.subcore_barrier()

        for kk in range(16):
            off = kk * 40
            cnt = 40 if off + 40 <= RPT else RPT - off
            pltpu.sync_copy(acc.at[pl.ds(r0 + off, cnt)],
                            zbuf.at[pl.ds(0, cnt)])
            pltpu.sync_copy(zbuf.at[pl.ds(0, cnt)],
                            agg_out.at[cid, m, pl.ds(r0 + off, cnt), :])

        @pl.when(cid == 0)
        def _():
            for off, cnt in zchd:
                pltpu.sync_copy(dacc.at[pl.ds(r0 + off, cnt)],
                                zd.at[pl.ds(0, cnt)])
                pltpu.sync_copy(zd.at[pl.ds(0, cnt)],
                                den_out.at[m, pl.ds(r0 + off, cnt), :])

        plsc.subcore_barrier()


def _dense_body(x_ref, w_ref, smat_ref, hph_ref, stab_ref):
    xb = x_ref[...]
    hp = jnp.dot(xb, w_ref[0], preferred_element_type=jnp.float32)
    hph_ref[0, 0] = hp[:, :FH]
    hph_ref[1, 0] = hp[:, FH:]
    stab_ref[0] = jnp.dot(hp, smat_ref[0], preferred_element_type=jnp.float32)


def _make_dense(din):
    return pl.pallas_call(
        _dense_body,
        grid=(M, NB),
        in_specs=[
            pl.BlockSpec((BN, din), lambda m, i: (i, 0)),
            pl.BlockSpec((1, din, F), lambda m, i: (m, 0, 0)),
            pl.BlockSpec((1, F, 16), lambda m, i: (m, 0, 0)),
        ],
        out_specs=[
            pl.BlockSpec((NC, 1, BN, FH), lambda m, i: (0, m, i, 0)),
            pl.BlockSpec((1, BN, 16), lambda m, i: (m, i, 0)),
        ],
        out_shape=[
            jax.ShapeDtypeStruct((NC, M, N, FH), jnp.float32),
            jax.ShapeDtypeStruct((M, N, 16), jnp.float32),
        ],
    )


_dense0 = _make_dense(DIN)
_dense1 = _make_dense(DH)


def _elu(v):
    return jnp.where(v > 0, v, jnp.exp(v) - 1.0)


def _mc_body(aggh_ref, den_ref, q_ref, h_ref):
    zs = []
    for m in range(M):
        acc = None
        for h in range(H):
            c, off = h // 2, (h % 2) * DH
            v = aggh_ref[c, m, :, off:off + DH] / (den_ref[m, :, h:h + 1] + 1e-9)
            ev = _elu(v)
            acc = ev if acc is None else acc + ev
        zs.append(acc * (1.0 / H))
    q = q_ref[...]
    ss = [jnp.sum(jnp.tanh(z) * q, axis=1, keepdims=True) for z in zs]
    smax = jnp.maximum(jnp.maximum(ss[0], ss[1]), ss[2])
    es = [jnp.exp(s - smax) for s in ss]
    tot = es[0] + es[1] + es[2]
    hsum = sum((e / tot) * z for e, z in zip(es, zs))
    h_ref[...] = jnp.maximum(hsum, 0.0)


_mc = pl.pallas_call(
    _mc_body,
    grid=(NB,),
    in_specs=[
        pl.BlockSpec((NC, M, BN, FH), lambda i: (0, 0, i, 0)),
        pl.BlockSpec((M, BN, H), lambda i: (0, i, 0)),
        pl.BlockSpec((1, DH), lambda i: (0, 0)),
    ],
    out_specs=pl.BlockSpec((BN, DH), lambda i: (i, 0)),
    out_shape=jax.ShapeDtypeStruct((N, DH), jnp.float32),
)


def _fc_body(aggh_ref, den_ref, wfc_ref, bfc_ref, out_ref):
    acc = jnp.zeros((BN, 16), jnp.float32) + bfc_ref[...]
    for m in range(M):
        for h in range(H):
            c, off = h // 2, (h % 2) * DH
            v = aggh_ref[c, m, :, off:off + DH] / (den_ref[m, :, h:h + 1] + 1e-9)
            ev = _elu(v)
            w = wfc_ref[(m * H + h) * DH:(m * H + h + 1) * DH, :]
            acc = acc + jnp.dot(ev, w, preferred_element_type=jnp.float32)
    out_ref[...] = acc


_fc = pl.pallas_call(
    _fc_body,
    grid=(NB,),
    in_specs=[
        pl.BlockSpec((NC, M, BN, FH), lambda i: (0, 0, i, 0)),
        pl.BlockSpec((M, BN, H), lambda i: (0, i, 0)),
        pl.BlockSpec((M * F, 16), lambda i: (0, 0)),
        pl.BlockSpec((1, 16), lambda i: (0, 0)),
    ],
    out_specs=pl.BlockSpec((BN, 16), lambda i: (i, 0)),
    out_shape=jax.ShapeDtypeStruct((N, 16), jnp.float32),
)


def _make_smat(a_src, a_dst):
    sm = jnp.zeros((M, F, 16), jnp.float32)
    for h in range(H):
        sm = sm.at[:, h * DH:(h + 1) * DH, h].set(a_src[:, h, :])
        sm = sm.at[:, h * DH:(h + 1) * DH, H + h].set(a_dst[:, h, :])
    return sm


def kernel(x, edge_index, W0, a_src0, a_dst0, attn_q, W1, a_src1, a_dst1,
           Wfc, bfc):
    Wr0 = jnp.transpose(W0, (0, 2, 1, 3)).reshape(M, DIN, F)
    Wr1 = jnp.transpose(W1, (0, 2, 1, 3)).reshape(M, DH, F)
    smat0 = _make_smat(a_src0, a_dst0)
    smat1 = _make_smat(a_src1, a_dst1)

    offs = (jnp.arange(M, dtype=jnp.int32) * N)[:, None]
    gsrc = edge_index[:, 0, :] + offs
    gdst = edge_index[:, 1, :] + offs
    gsrc2 = jnp.stack([gsrc, gsrc + M * N])
    dstl = edge_index[:, 1, :]

    z128 = jnp.zeros((40, FH), jnp.float32)
    z4 = jnp.zeros((160, H), jnp.float32)

    def layer(xin, Wr, smat, dense_fn):
        hph, stab = dense_fn(xin, Wr, smat)
        return _agg(hph.reshape(NC * M * N, FH), stab.reshape(M * N, 16),
                    gsrc, gsrc2, dstl, gdst, z128, z4)

    aggh0, den0 = layer(x, Wr0, smat0, _dense0)
    hmid = _mc(aggh0, den0, attn_q.reshape(1, DH))
    aggh1, den1 = layer(hmid, Wr1, smat1, _dense1)
    return _fc(aggh1, den1, Wfc, bfc.reshape(1, 16))
